# Initial kernel scaffold; baseline (speedup 1.0000x reference)
import jax
import jax.numpy as jnp
from jax.experimental import pallas as pl


def kernel(x, edge_index, W, b):
    raise NotImplementedError("write your pallas kernel here")


# speedup vs baseline: 24.6251x; 24.6251x over previous
import functools

import jax
import jax.numpy as jnp
from jax import lax
from jax.experimental import pallas as pl
from jax.experimental.pallas import tpu as pltpu
from jax.experimental.pallas import tpu_sc as plsc

NC = 2
NS = 16
K = 80
BN = 1000
V_TH = 1.0
T = 4


def _deg_body(dst_hbm, zeros_hbm, degp_hbm, idx_v, ones_v, deg_sh, sem):
    nchunk = dst_hbm.shape[2]
    c = lax.axis_index("c")
    s = lax.axis_index("s")

    @pl.when(s == 0)
    def _():
        pltpu.sync_copy(zeros_hbm, deg_sh)

    for i in range(K // 16):
        ones_v[pl.ds(i * 16, 16)] = jnp.ones((16,), jnp.float32)
    pltpu.sync_copy(dst_hbm.at[c, s], idx_v)
    plsc.subcore_barrier()

    def body(j, carry):
        pltpu.sync_copy(ones_v, deg_sh.at[idx_v.at[j]], add=True)
        return carry

    lax.fori_loop(0, nchunk, body, 0)
    plsc.subcore_barrier()

    @pl.when(s == 0)
    def _():
        pltpu.sync_copy(deg_sh, degp_hbm.at[c])


def _deg_partials(dst_r, zeros_n, n):
    nchunk = dst_r.shape[2]
    kern = pl.kernel(
        _deg_body,
        out_type=jax.ShapeDtypeStruct((NC, n), jnp.float32),
        mesh=plsc.VectorSubcoreMesh(core_axis_name="c", subcore_axis_name="s"),
        scratch_types=[
            pltpu.VMEM((nchunk, K), jnp.int32),
            pltpu.VMEM((K,), jnp.float32),
            pltpu.MemorySpace.VMEM_SHARED((n,), jnp.float32),
            pltpu.SemaphoreType.DMA,
        ],
    )
    return kern(dst_r, zeros_n)


def _scatter_body(g_hbm, src_hbm, dst_hbm, zeros_hbm, accp_hbm,
                  sidx_v, didx_v, rows_v, acc_sh, sem):
    nchunk = src_hbm.shape[2]
    n = g_hbm.shape[0]
    rows_per_tile = n // NS
    c = lax.axis_index("c")
    s = lax.axis_index("s")

    @pl.when(s == 0)
    def _():
        pltpu.sync_copy(zeros_hbm, acc_sh)

    pltpu.sync_copy(src_hbm.at[c, s], sidx_v)
    pltpu.sync_copy(dst_hbm.at[c, s], didx_v)
    plsc.subcore_barrier()

    def body(j, carry):
        pltpu.async_copy(g_hbm.at[sidx_v.at[j]], rows_v, sem).wait()
        pltpu.sync_copy(rows_v, acc_sh.at[didx_v.at[j]], add=True)
        return carry

    lax.fori_loop(0, nchunk, body, 0)
    plsc.subcore_barrier()

    @pl.when(s == 0)
    def _():
        pltpu.sync_copy(acc_sh, accp_hbm.at[c])


def _scatter_partials(g, src_r, dst_r, zeros_nd, n, d):
    nchunk = src_r.shape[2]
    kern = pl.kernel(
        _scatter_body,
        out_type=jax.ShapeDtypeStruct((NC, n, d), jnp.float32),
        mesh=plsc.VectorSubcoreMesh(core_axis_name="c", subcore_axis_name="s"),
        scratch_types=[
            pltpu.VMEM((nchunk, K), jnp.int32),
            pltpu.VMEM((nchunk, K), jnp.int32),
            pltpu.VMEM((K, d), jnp.float32),
            pltpu.MemorySpace.VMEM_SHARED((n, d), jnp.float32),
            pltpu.SemaphoreType.DMA,
        ],
    )
    return kern(g, src_r, dst_r, zeros_nd)


def _matmul_body(x_ref, w_ref, h_ref):
    h_ref[...] = jnp.dot(x_ref[...], w_ref[...],
                         preferred_element_type=jnp.float32)


def _matmul(x, w):
    n, din = x.shape
    dout = w.shape[1]
    return pl.pallas_call(
        _matmul_body,
        grid=(n // BN,),
        in_specs=[
            pl.BlockSpec((BN, din), lambda i: (i, 0)),
            pl.BlockSpec((din, dout), lambda i: (0, 0)),
        ],
        out_specs=pl.BlockSpec((BN, dout), lambda i: (i, 0)),
        out_shape=jax.ShapeDtypeStruct((n, dout), jnp.float32),
    )(x, w)


def _dinv_of(degp_blk):
    deg = degp_blk[:, 0:1] + degp_blk[:, 1:2] + 1.0
    return lax.rsqrt(jnp.maximum(deg, 1e-12))


def _scale_body(h_ref, degp_ref, g_ref):
    g_ref[...] = _dinv_of(degp_ref[...]) * h_ref[...]


def _scale(h, degp_t):
    n, d = h.shape
    return pl.pallas_call(
        _scale_body,
        grid=(n // BN,),
        in_specs=[
            pl.BlockSpec((BN, d), lambda i: (i, 0)),
            pl.BlockSpec((BN, NC), lambda i: (i, 0)),
        ],
        out_specs=pl.BlockSpec((BN, d), lambda i: (i, 0)),
        out_shape=jax.ShapeDtypeStruct((n, d), jnp.float32),
    )(h, degp_t)


def _if_body(accp_ref, g_ref, degp_ref, b_ref, o_ref, z_ref):
    dinv = _dinv_of(degp_ref[...])
    g = g_ref[...]
    y = dinv * (accp_ref[0] + accp_ref[1] + g) + b_ref[...]
    z = jnp.zeros_like(y)
    for t in range(T):
        z = z + y
        o = (z >= V_TH).astype(jnp.float32)
        z = z * (1.0 - o)
        o_ref[t] = o
        z_ref[t] = z


def _if_dynamics(accp, g, degp_t, b2d):
    n, d = g.shape
    out_sds = jax.ShapeDtypeStruct((T, n, d), jnp.float32)
    return pl.pallas_call(
        _if_body,
        grid=(n // BN,),
        in_specs=[
            pl.BlockSpec((NC, BN, d), lambda i: (0, i, 0)),
            pl.BlockSpec((BN, d), lambda i: (i, 0)),
            pl.BlockSpec((BN, NC), lambda i: (i, 0)),
            pl.BlockSpec((1, d), lambda i: (0, 0)),
        ],
        out_specs=[
            pl.BlockSpec((T, BN, d), lambda i: (0, i, 0)),
            pl.BlockSpec((T, BN, d), lambda i: (0, i, 0)),
        ],
        out_shape=[out_sds, out_sds],
    )(accp, g, degp_t, b2d)


def kernel(x, edge_index, W, b):
    n, din = x.shape
    dout = W.shape[1]
    e = edge_index.shape[1]
    ept = e // (NC * NS)
    nchunk = ept // K

    src_r = edge_index[0].reshape(NC, NS, nchunk, K)
    dst_r = edge_index[1].reshape(NC, NS, nchunk, K)
    zeros_n = jnp.zeros((n,), jnp.float32)
    zeros_nd = jnp.zeros((n, dout), jnp.float32)

    degp = _deg_partials(dst_r, zeros_n, n)
    degp_t = degp.T
    h = _matmul(x, W)
    g = _scale(h, degp_t)
    accp = _scatter_partials(g, src_r, dst_r, zeros_nd, n, dout)
    o_seq, z_seq = _if_dynamics(accp, g, degp_t, b.reshape(1, dout))
    return (o_seq, z_seq)
